# baseline (speedup 1.0000x reference)
import jax
import jax.numpy as jnp
from jax import lax
from jax.experimental import pallas as pl
from jax.experimental.pallas import tpu as pltpu
from jax.experimental.pallas import tpu_sc as plsc

_B, _S, _D = 4, 8192, 768
_B_TC = 3
_BS_TC = 512

_NC, _NS = 2, 16
_NW = _NC * _NS
_S_PER_W = _S // _NW
_CS = 16
_T = _S_PER_W // _CS
_LANES = 16
_GROUPS = _D // _LANES


def _tc_add_kernel(x_ref, t_ref, o_ref):
    o_ref[...] = x_ref[...] + t_ref[...]


def _tc_part(inputs, pos_table):
    return pl.pallas_call(
        _tc_add_kernel,
        grid=(_S // _BS_TC, _B_TC),
        in_specs=[
            pl.BlockSpec((1, _BS_TC, _D), lambda s, b: (b, s, 0)),
            pl.BlockSpec((_BS_TC, _D), lambda s, b: (s, 0)),
        ],
        out_specs=pl.BlockSpec((1, _BS_TC, _D), lambda s, b: (b, s, 0)),
        out_shape=jax.ShapeDtypeStruct((_B_TC, _S, _D), inputs.dtype),
    )(inputs, pos_table)


def _sc_body(in_hbm, tbl_hbm, out_hbm):
    def scoped(d0, d1, d2, t0, t1, t2,
               ls0, ls1, ls2, ss0, ss1, ss2):
        dbufs = (d0, d1, d2)
        tbls = (t0, t1, t2)
        lsems = (ls0, ls1, ls2)
        ssems = (ss0, ss1, ss2)

        wid = lax.axis_index("s") * _NC + lax.axis_index("c")
        s_base = wid * _S_PER_W

        def issue_load(t):
            i = t % 3
            off = s_base + t * _CS
            pltpu.async_copy(in_hbm.at[_B - 1, pl.ds(off, _CS)],
                             dbufs[i], lsems[i])
            pltpu.async_copy(tbl_hbm.at[pl.ds(off, _CS)], tbls[i], lsems[i])

        def issue_store(t):
            i = t % 3
            pltpu.async_copy(dbufs[i],
                             out_hbm.at[0, pl.ds(s_base + t * _CS, _CS)],
                             ssems[i])

        def wait_load(i):
            pltpu.make_async_copy(in_hbm.at[0, pl.ds(s_base, _CS)], dbufs[i],
                                  lsems[i]).wait()
            pltpu.make_async_copy(tbl_hbm.at[pl.ds(s_base, _CS)], tbls[i],
                                  lsems[i]).wait()

        def wait_store(i):
            pltpu.make_async_copy(dbufs[i], out_hbm.at[0, pl.ds(s_base, _CS)],
                                  ssems[i]).wait()

        def compute(i):
            def row(r, carry):
                for g in range(_GROUPS):
                    plsc.addupdate(
                        dbufs[i].at[r, pl.ds(g * _LANES, _LANES)],
                        tbls[i][r, pl.ds(g * _LANES, _LANES)],
                    )
                return carry
            lax.fori_loop(0, _CS, row, 0)

        issue_load(0)
        issue_load(1)

        for t in range(_T):
            i = t % 3
            if t >= 1:
                issue_store(t - 1)
            if t >= 2:
                wait_store((t - 2) % 3)
                if t + 1 < _T:
                    issue_load(t + 1)
            elif t == 1 and _T > 2:
                issue_load(2)
            wait_load(i)
            compute(i)

        issue_store(_T - 1)
        wait_store((_T - 2) % 3)
        wait_store((_T - 1) % 3)

    pl.run_scoped(
        scoped,
        *([pltpu.VMEM((_CS, _D), jnp.float32)] * 6
          + [pltpu.SemaphoreType.DMA] * 6),
    )


_sc_call = pl.kernel(
    _sc_body,
    out_type=jax.ShapeDtypeStruct((1, _S, _D), jnp.float32),
    mesh=plsc.VectorSubcoreMesh(core_axis_name="c", subcore_axis_name="s"),
)


def kernel(inputs, pos_table):
    tc_out = _tc_part(inputs, pos_table)
    sc_out = _sc_call(inputs, pos_table)
    return jnp.concatenate([tc_out, sc_out], axis=0)


# speedup vs baseline: 1.3546x; 1.3546x over previous
import jax
import jax.numpy as jnp
from jax import lax
from jax.experimental import pallas as pl
from jax.experimental.pallas import tpu as pltpu
from jax.experimental.pallas import tpu_sc as plsc

_B, _S, _D = 4, 8192, 768
_NC, _NS = 2, 16
_NW = _NC * _NS
_S_PER_W = _S // _NW
_CS = 32
_NCHUNK = _S_PER_W // _CS
_T = _NCHUNK * _B
_LANES = 16
_GROUPS = _D // _LANES


def _sc_body(in_hbm, tbl_hbm, out_hbm,
             d0, d1, d2, t0, t1,
             ls0, ls1, ls2, ss0, ss1, ss2, ts0, ts1):
    dbufs = (d0, d1, d2)
    tbls = (t0, t1)
    lsems = (ls0, ls1, ls2)
    ssems = (ss0, ss1, ss2)
    tsems = (ts0, ts1)

    wid = lax.axis_index("s") * _NC + lax.axis_index("c")
    s_base = wid * _S_PER_W

    def issue_load(t):
        c, b, i = t // _B, t % _B, t % 3
        pltpu.async_copy(in_hbm.at[b, pl.ds(s_base + c * _CS, _CS)],
                         dbufs[i], lsems[i])

    def issue_store(t):
        c, b, i = t // _B, t % _B, t % 3
        pltpu.async_copy(dbufs[i], out_hbm.at[b, pl.ds(s_base + c * _CS, _CS)],
                         ssems[i])

    def issue_tbl(c):
        j = c % 2
        pltpu.async_copy(tbl_hbm.at[pl.ds(s_base + c * _CS, _CS)],
                         tbls[j], tsems[j])

    def wait_load(i):
        pltpu.make_async_copy(in_hbm.at[0, pl.ds(s_base, _CS)], dbufs[i],
                              lsems[i]).wait()

    def wait_store(i):
        pltpu.make_async_copy(dbufs[i], out_hbm.at[0, pl.ds(s_base, _CS)],
                              ssems[i]).wait()

    def wait_tbl(j):
        pltpu.make_async_copy(tbl_hbm.at[pl.ds(s_base, _CS)], tbls[j],
                              tsems[j]).wait()

    def compute(i, j):
        def row(r, carry):
            for g in range(_GROUPS):
                plsc.addupdate(
                    dbufs[i].at[r, pl.ds(g * _LANES, _LANES)],
                    tbls[j][r, pl.ds(g * _LANES, _LANES)],
                )
            return carry
        lax.fori_loop(0, _CS, row, 0)

    issue_tbl(0)
    issue_tbl(1)
    for t in range(3):
        issue_load(t)

    for t in range(_T):
        c, b, i = t // _B, t % _B, t % 3
        if b == 0:
            wait_tbl(c % 2)
        wait_load(i)
        compute(i, c % 2)
        issue_store(t)
        if b == 3 and c + 2 < _NCHUNK:
            issue_tbl(c + 2)
        if t >= 1:
            wait_store((t - 1) % 3)
            if t + 2 < _T:
                issue_load(t + 2)

    wait_store((_T - 1) % 3)


_sc_call = pl.kernel(
    _sc_body,
    out_type=jax.ShapeDtypeStruct((_B, _S, _D), jnp.float32),
    mesh=plsc.VectorSubcoreMesh(core_axis_name="c", subcore_axis_name="s"),
    scratch_types=(
        [pltpu.VMEM((_CS, _D), jnp.float32)] * 3
        + [pltpu.VMEM((_CS, _D), jnp.float32)] * 2
        + [pltpu.SemaphoreType.DMA] * 8
    ),
)


def kernel(inputs, pos_table):
    return _sc_call(inputs, pos_table)
